# Initial kernel scaffold; baseline (speedup 1.0000x reference)
import jax
import jax.numpy as jnp
from jax.experimental import pallas as pl


def kernel(x, edge_index, batch, W_conv, b_conv, W_fc, b_fc):
    raise NotImplementedError("write your pallas kernel here")


# speedup vs baseline: 4.4996x; 4.4996x over previous
import functools

import jax
import jax.numpy as jnp
from jax import lax
from jax.experimental import pallas as pl
from jax.experimental.pallas import tpu as pltpu
from jax.experimental.pallas import tpu_sc as plsc

N_NODES = 10000
N_EDGES = 320000
D = 128
N_CLASS = 10
N_GRAPHS = 64

NW = 32
ROWS_PAD = 10240
ROWS_PER_TILE = ROWS_PAD // 16
E_PER_TILE = 10240
E_PAD = NW * E_PER_TILE
CHUNK = 128
N_CHUNKS = E_PER_TILE // CHUNK
DEG_ROWS = ROWS_PAD // 128


def _sc_body(src_hbm, dst_hbm, x_hbm, zag_hbm,
             parts_out, deg_out,
             src_v, dst_v, msg_v, hist_v, agg_sh, sem):
    c = lax.axis_index("c")
    s = lax.axis_index("s")
    wid = s * 2 + c

    rbase = s * ROWS_PER_TILE
    for j in range(ROWS_PER_TILE // CHUNK):
        pltpu.sync_copy(zag_hbm, agg_sh.at[pl.ds(rbase + j * CHUNK, CHUNK)])
    pltpu.sync_copy(zag_hbm.at[pl.ds(0, DEG_ROWS)], hist_v)
    plsc.subcore_barrier()

    ebase = wid * E_PER_TILE
    ones16 = jnp.ones((16,), jnp.float32)

    def chunk_body(j, carry):
        base = ebase + j * CHUNK
        pltpu.sync_copy(src_hbm.at[pl.ds(base, CHUNK)], src_v)
        pltpu.sync_copy(dst_hbm.at[pl.ds(base, CHUNK)], dst_v)
        pltpu.async_copy(x_hbm.at[src_v], msg_v, sem).wait()
        pltpu.sync_copy(msg_v, agg_sh.at[dst_v], add=True)
        for g in range(CHUNK // 16):
            idx16 = dst_v[pl.ds(g * 16, 16)]
            r16 = jnp.right_shift(idx16, 7)
            c16 = jnp.bitwise_and(idx16, 127)
            plsc.addupdate_scatter(hist_v, [r16, c16], ones16)
        return carry

    lax.fori_loop(0, N_CHUNKS, chunk_body, 0)
    plsc.subcore_barrier()

    for j in range(ROWS_PER_TILE // CHUNK):
        pltpu.sync_copy(agg_sh.at[pl.ds(rbase + j * CHUNK, CHUNK)],
                        parts_out.at[c, pl.ds(rbase + j * CHUNK, CHUNK)])
    pltpu.sync_copy(hist_v, deg_out.at[wid])


@functools.cache
def _make_sc_scatter():
    mesh = plsc.VectorSubcoreMesh(core_axis_name="c", subcore_axis_name="s",
                                  num_cores=2, num_subcores=16)
    return pl.kernel(
        _sc_body,
        out_type=[
            jax.ShapeDtypeStruct((2, ROWS_PAD, D), jnp.float32),
            jax.ShapeDtypeStruct((NW, DEG_ROWS, 128), jnp.float32),
        ],
        mesh=mesh,
        scratch_types=[
            pltpu.VMEM((CHUNK,), jnp.int32),
            pltpu.VMEM((CHUNK,), jnp.int32),
            pltpu.VMEM((CHUNK, D), jnp.float32),
            pltpu.VMEM((DEG_ROWS, 128), jnp.float32),
            pltpu.VMEM_SHARED((ROWS_PAD, D), jnp.float32),
            pltpu.SemaphoreType.DMA,
        ],
        compiler_params=pltpu.CompilerParams(needs_layout_passes=False),
    )


ROW_BLK = 1024
N_BLK = ROWS_PAD // ROW_BLK
SUB = 128
N_SUB = ROW_BLK // SUB


def _tc_body(parts_ref, deg_ref, batch_ref, wc_ref, bc_ref, wf_ref, bf_ref,
             out_ref, gsum, cnt):
    i = pl.program_id(0)

    @pl.when(i == 0)
    def _():
        gsum[...] = jnp.zeros_like(gsum)
        cnt[...] = jnp.zeros_like(cnt)

    p = parts_ref[0] + parts_ref[1]
    dsum = deg_ref[0]
    for t in range(1, NW):
        dsum = dsum + deg_ref[t]
    dsum = jnp.maximum(dsum, 1.0)

    for k in range(N_SUB):
        pk = p[k * SUB:(k + 1) * SUB, :]
        hT = lax.dot_general(wc_ref[...], pk, (((0,), (1,)), ((), ())),
                             preferred_element_type=jnp.float32)
        hT = hT / dsum[k:k + 1, :]
        hT = jnp.maximum(hT + bc_ref[...], 0.0)
        bk = batch_ref[pl.ds(k * SUB, SUB), :]
        ohk = (bk == lax.broadcasted_iota(jnp.int32, (SUB, N_GRAPHS), 1)
               ).astype(jnp.float32)
        gsum[...] += lax.dot_general(hT, ohk, (((1,), (0,)), ((), ())),
                                     preferred_element_type=jnp.float32)
        cnt[...] += jnp.sum(ohk, axis=0, keepdims=True)

    @pl.when(i == N_BLK - 1)
    def _():
        gT = gsum[...] / jnp.maximum(cnt[...], 1.0)
        out_ref[...] = lax.dot_general(
            gT, wf_ref[...], (((0,), (0,)), ((), ())),
            preferred_element_type=jnp.float32) + bf_ref[...]


_tc_finish = pl.pallas_call(
    _tc_body,
    grid=(N_BLK,),
    in_specs=[
        pl.BlockSpec((2, ROW_BLK, D), lambda i: (0, i, 0)),
        pl.BlockSpec((NW, ROW_BLK // 128, 128), lambda i: (0, i, 0)),
        pl.BlockSpec((ROW_BLK, 1), lambda i: (i, 0)),
        pl.BlockSpec((D, D), lambda i: (0, 0)),
        pl.BlockSpec((D, 1), lambda i: (0, 0)),
        pl.BlockSpec((D, N_CLASS), lambda i: (0, 0)),
        pl.BlockSpec((1, N_CLASS), lambda i: (0, 0)),
    ],
    out_specs=pl.BlockSpec((N_GRAPHS, N_CLASS), lambda i: (0, 0)),
    out_shape=jax.ShapeDtypeStruct((N_GRAPHS, N_CLASS), jnp.float32),
    scratch_shapes=[
        pltpu.VMEM((D, N_GRAPHS), jnp.float32),
        pltpu.VMEM((1, N_GRAPHS), jnp.float32),
    ],
)


def kernel(x, edge_index, batch, W_conv, b_conv, W_fc, b_fc):
    src = edge_index[0].astype(jnp.int32)
    dst = edge_index[1].astype(jnp.int32)
    pad = E_PAD - N_EDGES
    src = jnp.concatenate([src, jnp.zeros((pad,), jnp.int32)])
    dst = jnp.concatenate([dst, jnp.full((pad,), N_NODES, jnp.int32)])
    zag = jnp.zeros((CHUNK, D), jnp.float32)

    parts, degw = _make_sc_scatter()(src, dst, x, zag)

    batch_pad = jnp.concatenate(
        [batch.astype(jnp.int32),
         jnp.full((ROWS_PAD - N_NODES,), N_GRAPHS, jnp.int32)]
    ).reshape(ROWS_PAD, 1)
    return _tc_finish(parts, degw, batch_pad, W_conv,
                      b_conv.reshape(D, 1), W_fc, b_fc.reshape(1, N_CLASS))


# speedup vs baseline: 4.9876x; 1.1085x over previous
import functools

import jax
import jax.numpy as jnp
from jax import lax
from jax.experimental import pallas as pl
from jax.experimental.pallas import tpu as pltpu
from jax.experimental.pallas import tpu_sc as plsc

N_NODES = 10000
N_EDGES = 320000
D = 128
N_CLASS = 10
N_GRAPHS = 64

NW = 32
ROWS_PAD = 10240
ROWS_PER_TILE = ROWS_PAD // 16
E_PER_TILE = 10240
E_PAD = NW * E_PER_TILE
CHUNK = 128
N_CHUNKS = E_PER_TILE // CHUNK
DEG_ROWS = ROWS_PAD // 128


def _sc_body(src_hbm, dst_hbm, x_hbm, zag_hbm,
             parts_out, deg_out,
             src0, src1, dst0, dst1, msg0, msg1, hist_v, agg_sh, sem0, sem1):
    c = lax.axis_index("c")
    s = lax.axis_index("s")
    wid = s * 2 + c

    rbase = s * ROWS_PER_TILE
    for j in range(ROWS_PER_TILE // CHUNK):
        pltpu.sync_copy(zag_hbm, agg_sh.at[pl.ds(rbase + j * CHUNK, CHUNK)])
    pltpu.sync_copy(zag_hbm.at[pl.ds(0, DEG_ROWS)], hist_v)
    plsc.subcore_barrier()

    ones16 = jnp.ones((16,), jnp.float32)
    srcs = (src0, src1)
    dsts = (dst0, dst1)
    msgs = (msg0, msg1)
    sems = (sem0, sem1)
    ebase = wid * E_PER_TILE

    def load_idx(cj, b):
        base = ebase + cj * CHUNK
        pltpu.sync_copy(src_hbm.at[pl.ds(base, CHUNK)], srcs[b])
        pltpu.sync_copy(dst_hbm.at[pl.ds(base, CHUNK)], dsts[b])

    def scatter_chunk(b):
        pltpu.sync_copy(msgs[b], agg_sh.at[dsts[b]], add=True)
        for g in range(CHUNK // 16):
            idx16 = dsts[b][pl.ds(g * 16, 16)]
            r16 = jnp.right_shift(idx16, 7)
            c16 = jnp.bitwise_and(idx16, 127)
            plsc.addupdate_scatter(hist_v, [r16, c16], ones16)

    load_idx(0, 0)
    pltpu.async_copy(x_hbm.at[src0], msg0, sem0)

    def pair_body(jp, carry):
        for b in range(2):
            cj = jp * 2 + b
            if b == 0:
                @pl.when(jp > 0)
                def _():
                    scatter_chunk(1)
                load_idx(cj + 1, 1)
            else:
                scatter_chunk(0)

                @pl.when(jp < N_CHUNKS // 2 - 1)
                def _():
                    load_idx(cj + 1, 0)
            pltpu.make_async_copy(x_hbm.at[srcs[b]], msgs[b], sems[b]).wait()
            if b == 0:
                pltpu.async_copy(x_hbm.at[src1], msg1, sem1)
            else:
                @pl.when(jp < N_CHUNKS // 2 - 1)
                def _():
                    pltpu.async_copy(x_hbm.at[src0], msg0, sem0)
        return carry

    lax.fori_loop(0, N_CHUNKS // 2, pair_body, 0)
    scatter_chunk(1)
    plsc.subcore_barrier()

    for j in range(ROWS_PER_TILE // CHUNK):
        pltpu.sync_copy(agg_sh.at[pl.ds(rbase + j * CHUNK, CHUNK)],
                        parts_out.at[c, pl.ds(rbase + j * CHUNK, CHUNK)])
    pltpu.sync_copy(hist_v, deg_out.at[wid])


@functools.cache
def _make_sc_scatter():
    mesh = plsc.VectorSubcoreMesh(core_axis_name="c", subcore_axis_name="s",
                                  num_cores=2, num_subcores=16)
    return pl.kernel(
        _sc_body,
        out_type=[
            jax.ShapeDtypeStruct((2, ROWS_PAD, D), jnp.float32),
            jax.ShapeDtypeStruct((NW, DEG_ROWS, 128), jnp.float32),
        ],
        mesh=mesh,
        scratch_types=[
            pltpu.VMEM((CHUNK,), jnp.int32),
            pltpu.VMEM((CHUNK,), jnp.int32),
            pltpu.VMEM((CHUNK,), jnp.int32),
            pltpu.VMEM((CHUNK,), jnp.int32),
            pltpu.VMEM((CHUNK, D), jnp.float32),
            pltpu.VMEM((CHUNK, D), jnp.float32),
            pltpu.VMEM((DEG_ROWS, 128), jnp.float32),
            pltpu.VMEM_SHARED((ROWS_PAD, D), jnp.float32),
            pltpu.SemaphoreType.DMA,
            pltpu.SemaphoreType.DMA,
        ],
        compiler_params=pltpu.CompilerParams(needs_layout_passes=False),
    )


ROW_BLK = 1024
N_BLK = ROWS_PAD // ROW_BLK
SUB = 128
N_SUB = ROW_BLK // SUB


def _tc_body(parts_ref, deg_ref, batch_ref, wc_ref, bc_ref, wf_ref, bf_ref,
             out_ref, gsum, cnt):
    i = pl.program_id(0)

    @pl.when(i == 0)
    def _():
        gsum[...] = jnp.zeros_like(gsum)
        cnt[...] = jnp.zeros_like(cnt)

    p = parts_ref[0] + parts_ref[1]
    dsum = deg_ref[0]
    for t in range(1, NW):
        dsum = dsum + deg_ref[t]
    dsum = jnp.maximum(dsum, 1.0)

    for k in range(N_SUB):
        pk = p[k * SUB:(k + 1) * SUB, :]
        hT = lax.dot_general(wc_ref[...], pk, (((0,), (1,)), ((), ())),
                             preferred_element_type=jnp.float32)
        hT = hT / dsum[k:k + 1, :]
        hT = jnp.maximum(hT + bc_ref[...], 0.0)
        bk = batch_ref[pl.ds(k * SUB, SUB), :]
        ohk = (bk == lax.broadcasted_iota(jnp.int32, (SUB, N_GRAPHS), 1)
               ).astype(jnp.float32)
        gsum[...] += lax.dot_general(hT, ohk, (((1,), (0,)), ((), ())),
                                     preferred_element_type=jnp.float32)
        cnt[...] += jnp.sum(ohk, axis=0, keepdims=True)

    @pl.when(i == N_BLK - 1)
    def _():
        gT = gsum[...] / jnp.maximum(cnt[...], 1.0)
        out_ref[...] = lax.dot_general(
            gT, wf_ref[...], (((0,), (0,)), ((), ())),
            preferred_element_type=jnp.float32) + bf_ref[...]


_tc_finish = pl.pallas_call(
    _tc_body,
    grid=(N_BLK,),
    in_specs=[
        pl.BlockSpec((2, ROW_BLK, D), lambda i: (0, i, 0)),
        pl.BlockSpec((NW, ROW_BLK // 128, 128), lambda i: (0, i, 0)),
        pl.BlockSpec((ROW_BLK, 1), lambda i: (i, 0)),
        pl.BlockSpec((D, D), lambda i: (0, 0)),
        pl.BlockSpec((D, 1), lambda i: (0, 0)),
        pl.BlockSpec((D, N_CLASS), lambda i: (0, 0)),
        pl.BlockSpec((1, N_CLASS), lambda i: (0, 0)),
    ],
    out_specs=pl.BlockSpec((N_GRAPHS, N_CLASS), lambda i: (0, 0)),
    out_shape=jax.ShapeDtypeStruct((N_GRAPHS, N_CLASS), jnp.float32),
    scratch_shapes=[
        pltpu.VMEM((D, N_GRAPHS), jnp.float32),
        pltpu.VMEM((1, N_GRAPHS), jnp.float32),
    ],
)


def kernel(x, edge_index, batch, W_conv, b_conv, W_fc, b_fc):
    src = edge_index[0].astype(jnp.int32)
    dst = edge_index[1].astype(jnp.int32)
    pad = E_PAD - N_EDGES
    src = jnp.concatenate([src, jnp.zeros((pad,), jnp.int32)])
    dst = jnp.concatenate([dst, jnp.full((pad,), N_NODES, jnp.int32)])
    zag = jnp.zeros((CHUNK, D), jnp.float32)

    parts, degw = _make_sc_scatter()(src, dst, x, zag)

    batch_pad = jnp.concatenate(
        [batch.astype(jnp.int32),
         jnp.full((ROWS_PAD - N_NODES,), N_GRAPHS, jnp.int32)]
    ).reshape(ROWS_PAD, 1)
    return _tc_finish(parts, degw, batch_pad, W_conv,
                      b_conv.reshape(D, 1), W_fc, b_fc.reshape(1, N_CLASS))
